# baseline (speedup 1.0000x reference)
import functools

import jax
import jax.numpy as jnp
from jax import lax
from jax.experimental import pallas as pl
from jax.experimental.pallas import tpu as pltpu
from jax.experimental.pallas import tpu_sc as plsc


H = 128
NC = 2
NS = 16
NW = NC * NS

K = 4
BIN = 12544
NPAD = K * BIN
TRASH = BIN
B = 192
STRIPE = BIN // NS

TC_BLK = NPAD // 16


def _linrelu_body(x_ref, w_ref, b_ref, o_ref):
    o_ref[...] = jax.nn.relu(
        jnp.dot(x_ref[...], w_ref[...], preferred_element_type=jnp.float32)
        + b_ref[...]
    )


def _linrelu(x, w, b):
    d = x.shape[1]
    h = w.shape[1]
    return pl.pallas_call(
        _linrelu_body,
        grid=(NPAD // TC_BLK,),
        in_specs=[
            pl.BlockSpec((TC_BLK, d), lambda i: (i, 0)),
            pl.BlockSpec((d, h), lambda i: (0, 0)),
            pl.BlockSpec((h,), lambda i: (0,)),
        ],
        out_specs=pl.BlockSpec((TC_BLK, h), lambda i: (i, 0)),
        out_shape=jax.ShapeDtypeStruct((NPAD, h), jnp.float32),
    )(x, w, b)


def _sage_linear_body(s_ref, c_ref, xd_ref, wl_ref, bl_ref, wr_ref, o_ref):
    rec = 1.0 / jnp.maximum(c_ref[...][:, :1], 1.0)
    mean = s_ref[...] * rec
    o_ref[...] = jax.nn.relu(
        jnp.dot(mean, wl_ref[...], preferred_element_type=jnp.float32)
        + bl_ref[...]
        + jnp.dot(xd_ref[...], wr_ref[...], preferred_element_type=jnp.float32)
    )


def _sage_linear(s, cnt, x_dst, wl, bl, wr):
    return pl.pallas_call(
        _sage_linear_body,
        grid=(NPAD // TC_BLK,),
        in_specs=[
            pl.BlockSpec((TC_BLK, H), lambda i: (i, 0)),
            pl.BlockSpec((TC_BLK, H), lambda i: (i, 0)),
            pl.BlockSpec((TC_BLK, H), lambda i: (i, 0)),
            pl.BlockSpec((H, H), lambda i: (0, 0)),
            pl.BlockSpec((H,), lambda i: (0,)),
            pl.BlockSpec((H, H), lambda i: (0, 0)),
        ],
        out_specs=pl.BlockSpec((TC_BLK, H), lambda i: (i, 0)),
        out_shape=jax.ShapeDtypeStruct((NPAD, H), jnp.float32),
    )(s, cnt, x_dst, wl, bl, wr)


def _sage_linear2_body(s_ref, c_ref, xd_ref, wl_ref, bl_ref, wr_ref, wa_ref,
                       o_ref, a_ref):
    rec = 1.0 / jnp.maximum(c_ref[...][:, :1], 1.0)
    mean = s_ref[...] * rec
    h2 = jax.nn.relu(
        jnp.dot(mean, wl_ref[...], preferred_element_type=jnp.float32)
        + bl_ref[...]
        + jnp.dot(xd_ref[...], wr_ref[...], preferred_element_type=jnp.float32)
    )
    o_ref[...] = h2
    a_ref[...] = jnp.dot(h2, wa_ref[...], preferred_element_type=jnp.float32)


def _sage_linear2(s, cnt, x_dst, wl, bl, wr, wa):
    return pl.pallas_call(
        _sage_linear2_body,
        grid=(NPAD // TC_BLK,),
        in_specs=[
            pl.BlockSpec((TC_BLK, H), lambda i: (i, 0)),
            pl.BlockSpec((TC_BLK, H), lambda i: (i, 0)),
            pl.BlockSpec((TC_BLK, H), lambda i: (i, 0)),
            pl.BlockSpec((H, H), lambda i: (0, 0)),
            pl.BlockSpec((H,), lambda i: (0,)),
            pl.BlockSpec((H, H), lambda i: (0, 0)),
            pl.BlockSpec((H, H), lambda i: (0, 0)),
        ],
        out_specs=[
            pl.BlockSpec((TC_BLK, H), lambda i: (i, 0)),
            pl.BlockSpec((TC_BLK, H), lambda i: (i, 0)),
        ],
        out_shape=[
            jax.ShapeDtypeStruct((NPAD, H), jnp.float32),
            jax.ShapeDtypeStruct((NPAD, H), jnp.float32),
        ],
    )(s, cnt, x_dst, wl, bl, wr, wa)


def _final_body(gu_ref, gm_ref, ea_ref, we_ref, be_ref, w1c_ref, b1_ref,
                w2_ref, b2_ref, o_ref):
    e = jax.nn.relu(
        jnp.dot(ea_ref[...], we_ref[...], preferred_element_type=jnp.float32)
        + be_ref[...]
    )
    acc = gu_ref[...] + gm_ref[...] + jnp.dot(
        e, w1c_ref[...], preferred_element_type=jnp.float32)
    h = jax.nn.relu(acc + b1_ref[...])
    o_ref[...] = (
        jnp.dot(h, w2_ref[...], preferred_element_type=jnp.float32) + b2_ref[...]
    )


def _final(gu, gm, ea, we, be, w1c, b1, w2, b2, block):
    n = gu.shape[0]
    d_e = ea.shape[1]
    return pl.pallas_call(
        _final_body,
        grid=(n // block,),
        in_specs=[
            pl.BlockSpec((block, H), lambda i: (i, 0)),
            pl.BlockSpec((block, H), lambda i: (i, 0)),
            pl.BlockSpec((block, d_e), lambda i: (i, 0)),
            pl.BlockSpec((d_e, H), lambda i: (0, 0)),
            pl.BlockSpec((H,), lambda i: (0,)),
            pl.BlockSpec((H, H), lambda i: (0, 0)),
            pl.BlockSpec((H,), lambda i: (0,)),
            pl.BlockSpec((H, 2), lambda i: (0, 0)),
            pl.BlockSpec((2,), lambda i: (0,)),
        ],
        out_specs=pl.BlockSpec((block, 2), lambda i: (i, 0)),
        out_shape=jax.ShapeDtypeStruct((n, 2), jnp.float32),
    )(gu, gm, ea, we, be, w1c, b1, w2, b2)


def _sc_mesh():
    return plsc.VectorSubcoreMesh(core_axis_name="c", subcore_axis_name="s",
                                  num_cores=NC, num_subcores=NS)


def _bin_edges(ei_src, ei_dst):
    e_tot = ei_src.shape[0]
    epw = e_tot // NW
    cap = -(-epw // B) * B + 16

    @functools.partial(
        pl.kernel,
        out_type=[
            jax.ShapeDtypeStruct((K * NW * cap,), jnp.int32),
            jax.ShapeDtypeStruct((K * NW * cap,), jnp.int32),
            jax.ShapeDtypeStruct((NW * 16,), jnp.int32),
        ],
        mesh=_sc_mesh(),
        compiler_params=pltpu.CompilerParams(needs_layout_passes=False),
        scratch_types=[
            pltpu.VMEM((epw,), jnp.int32),
            pltpu.VMEM((epw,), jnp.int32),
            [pltpu.VMEM((cap,), jnp.int32) for _ in range(K)],
            [pltpu.VMEM((cap,), jnp.int32) for _ in range(K)],
            pltpu.VMEM((16,), jnp.int32),
        ],
    )
    def k(es_hbm, ed_hbm, srcl_hbm, dstl_hbm, cnts_hbm,
          srcbuf, dstbuf, sbufs, dbufs, countbuf):
        wid = lax.axis_index("s") * NC + lax.axis_index("c")
        lane = lax.iota(jnp.int32, 16)
        pltpu.sync_copy(es_hbm.at[pl.ds(wid * epw, epw)], srcbuf)
        pltpu.sync_copy(ed_hbm.at[pl.ds(wid * epw, epw)], dstbuf)

        def step(i, offs):
            vs = srcbuf[pl.ds(i * 16, 16)]
            vd = dstbuf[pl.ds(i * 16, 16)]
            binv = ((vd >= BIN).astype(jnp.int32)
                    + (vd >= 2 * BIN).astype(jnp.int32)
                    + (vd >= 3 * BIN).astype(jnp.int32))
            dstl = vd - binv * BIN
            new = []
            for kk in range(K):
                m = binv == kk
                o = offs[kk]
                plsc.store_compressed(sbufs[kk].at[pl.ds(o, 16)], vs, mask=m)
                plsc.store_compressed(dbufs[kk].at[pl.ds(o, 16)], dstl, mask=m)
                pc = plsc.all_reduce_population_count(m)
                new.append(o + pc[0])
            return tuple(new)

        offs = lax.fori_loop(0, epw // 16, step, (0, 0, 0, 0))

        cvec = jnp.zeros((16,), jnp.int32)
        for kk in range(K):
            n = offs[kk]
            base = (n >> 4) << 4
            npad = ((n + (B - 1)) // B) * B
            keep = lane < (n - base)
            vs_old = sbufs[kk][pl.ds(base, 16)]
            vd_old = dbufs[kk][pl.ds(base, 16)]
            sbufs[kk][pl.ds(base, 16)] = jnp.where(keep, vs_old, 0)
            dbufs[kk][pl.ds(base, 16)] = jnp.where(
                keep, vd_old, jnp.full((16,), TRASH, jnp.int32))

            def pad_step(j, carry, kk=kk):
                sbufs[kk][pl.ds(j * 16, 16)] = jnp.zeros((16,), jnp.int32)
                dbufs[kk][pl.ds(j * 16, 16)] = jnp.full((16,), TRASH,
                                                        jnp.int32)
                return carry

            lax.fori_loop((base >> 4) + 1, npad >> 4, pad_step, 0)
            cvec = jnp.where(lane == kk, npad // B, cvec)
            off_hbm = (kk * NW + wid) * cap
            pltpu.sync_copy(sbufs[kk].at[pl.ds(0, cap)],
                            srcl_hbm.at[pl.ds(off_hbm, cap)])
            pltpu.sync_copy(dbufs[kk].at[pl.ds(0, cap)],
                            dstl_hbm.at[pl.ds(off_hbm, cap)])
        countbuf[...] = cvec
        pltpu.sync_copy(countbuf, cnts_hbm.at[pl.ds(wid * 16, 16)])

    return k(ei_src, ei_dst)


def _sage_agg(h_src, src_list, dstl_list, counts, zeros_acc, cap):
    @functools.partial(
        pl.kernel,
        out_type=jax.ShapeDtypeStruct((NPAD, H), jnp.float32),
        mesh=_sc_mesh(),
        compiler_params=pltpu.CompilerParams(needs_layout_passes=False),
        scratch_types=[
            pltpu.VMEM((8 * B,), jnp.int32),
            pltpu.VMEM((8 * B,), jnp.int32),
            pltpu.VMEM((B, H), jnp.float32),
            pltpu.VMEM((NW * 16,), jnp.int32),
            pltpu.VMEM_SHARED((BIN + 16, H), jnp.float32),
            pltpu.SemaphoreType.DMA,
        ],
    )
    def k(h_hbm, srcl_hbm, dstl_hbm, cnts_hbm, zacc_hbm, s_hbm,
          idx0, dl0, gb0, cbuf, acc, sem0):
        c = lax.axis_index("c")
        sid = lax.axis_index("s")
        lane = lax.iota(jnp.int32, 16)
        pltpu.sync_copy(cnts_hbm, cbuf)
        sbase = sid * STRIPE

        for ki in range(2):
            kbin = c * 2 + ki
            pltpu.sync_copy(zacc_hbm, acc.at[pl.ds(sbase, STRIPE)])
            plsc.subcore_barrier()
            for li in range(2):
                st = sid * 2 + li
                cvec = cbuf[pl.ds(st * 16, 16)]
                trips = jnp.max(jnp.where(lane == kbin, cvec, 0))
                listbase = (kbin * NW + st) * cap

                def superbatch(sb, carry):
                    off = listbase + sb * (8 * B)
                    pltpu.sync_copy(srcl_hbm.at[pl.ds(off, 8 * B)], idx0)
                    pltpu.sync_copy(dstl_hbm.at[pl.ds(off, 8 * B)], dl0)
                    for t in range(8):
                        pltpu.async_copy(
                            h_hbm.at[idx0.at[pl.ds(t * B, B)]], gb0,
                            sem0).wait()
                        pltpu.sync_copy(gb0,
                                        acc.at[dl0.at[pl.ds(t * B, B)]],
                                        add=True)
                    return carry

                lax.fori_loop(0, trips // 8, superbatch, 0)

                def tail(j, carry):
                    off = listbase + j * B
                    pltpu.sync_copy(srcl_hbm.at[pl.ds(off, B)],
                                    idx0.at[pl.ds(0, B)])
                    pltpu.sync_copy(dstl_hbm.at[pl.ds(off, B)],
                                    dl0.at[pl.ds(0, B)])
                    pltpu.async_copy(h_hbm.at[idx0.at[pl.ds(0, B)]], gb0,
                                     sem0).wait()
                    pltpu.sync_copy(gb0, acc.at[dl0.at[pl.ds(0, B)]],
                                    add=True)
                    return carry

                lax.fori_loop((trips // 8) * 8, trips, tail, 0)
            plsc.subcore_barrier()
            pltpu.sync_copy(acc.at[pl.ds(sbase, STRIPE)],
                            s_hbm.at[pl.ds(kbin * BIN + sbase, STRIPE)])
            plsc.subcore_barrier()

    return k(h_src, src_list, dstl_list, counts, zeros_acc)


def _seg_counts(dstl_list, counts, zeros_cnt, ones_b, cap):
    @functools.partial(
        pl.kernel,
        out_type=jax.ShapeDtypeStruct((NPAD, H), jnp.float32),
        mesh=_sc_mesh(),
        compiler_params=pltpu.CompilerParams(needs_layout_passes=False),
        scratch_types=[
            pltpu.VMEM((8 * B,), jnp.int32),
            pltpu.VMEM((B, H), jnp.float32),
            pltpu.VMEM((NW * 16,), jnp.int32),
            pltpu.VMEM_SHARED((BIN + 16, H), jnp.float32),
        ],
    )
    def k(dstl_hbm, cnts_hbm, zcnt_hbm, ones_hbm,
          cnt_hbm, dstlbuf, ones_v, cbuf, cacc):
        c = lax.axis_index("c")
        sid = lax.axis_index("s")
        lane = lax.iota(jnp.int32, 16)
        pltpu.sync_copy(cnts_hbm, cbuf)
        pltpu.sync_copy(ones_hbm, ones_v)
        sbase = sid * STRIPE

        for ki in range(2):
            kbin = c * 2 + ki
            pltpu.sync_copy(zcnt_hbm, cacc.at[pl.ds(sbase, STRIPE)])
            plsc.subcore_barrier()
            for li in range(2):
                st = sid * 2 + li
                cvec = cbuf[pl.ds(st * 16, 16)]
                trips = jnp.max(jnp.where(lane == kbin, cvec, 0))
                listbase = (kbin * NW + st) * cap

                def superbatch(sb, carry):
                    off = listbase + sb * (8 * B)
                    pltpu.sync_copy(dstl_hbm.at[pl.ds(off, 8 * B)], dstlbuf)
                    for t in range(8):
                        pltpu.sync_copy(ones_v,
                                        cacc.at[dstlbuf.at[pl.ds(t * B, B)]],
                                        add=True)
                    return carry

                lax.fori_loop(0, trips // 8, superbatch, 0)

                def tail(j, carry):
                    off = listbase + j * B
                    pltpu.sync_copy(dstl_hbm.at[pl.ds(off, B)],
                                    dstlbuf.at[pl.ds(0, B)])
                    pltpu.sync_copy(ones_v,
                                    cacc.at[dstlbuf.at[pl.ds(0, B)]],
                                    add=True)
                    return carry

                lax.fori_loop((trips // 8) * 8, trips, tail, 0)
            plsc.subcore_barrier()
            pltpu.sync_copy(cacc.at[pl.ds(sbase, STRIPE)],
                            cnt_hbm.at[pl.ds(kbin * BIN + sbase, STRIPE)])
            plsc.subcore_barrier()

    return k(dstl_list, counts, zeros_cnt, ones_b)


def _head_gather(a_u, a_m, ei_u, ei_m):
    e_tot = ei_u.shape[0]
    per_w = e_tot // NW
    bb = 400
    sbt = 5
    trips = per_w // bb
    assert trips % sbt == 0 and per_w % bb == 0

    @functools.partial(
        pl.kernel,
        out_type=[
            jax.ShapeDtypeStruct((e_tot, H), jnp.float32),
            jax.ShapeDtypeStruct((e_tot, H), jnp.float32),
        ],
        mesh=_sc_mesh(),
        compiler_params=pltpu.CompilerParams(needs_layout_passes=False),
        scratch_types=[
            pltpu.VMEM((sbt * bb,), jnp.int32),
            pltpu.VMEM((sbt * bb,), jnp.int32),
            pltpu.VMEM((bb, H), jnp.float32),
            pltpu.VMEM((bb, H), jnp.float32),
            pltpu.SemaphoreType.DMA,
            pltpu.SemaphoreType.DMA,
        ],
    )
    def k(au_hbm, am_hbm, eiu_hbm, eim_hbm, gu_hbm, gm_hbm,
          idx_u, idx_m, buf_u, buf_m, sem_u, sem_m):
        wid = lax.axis_index("s") * NC + lax.axis_index("c")
        base_w = wid * per_w

        def superbatch(sb, carry):
            base = base_w + sb * (sbt * bb)
            pltpu.sync_copy(eiu_hbm.at[pl.ds(base, sbt * bb)], idx_u)
            pltpu.sync_copy(eim_hbm.at[pl.ds(base, sbt * bb)], idx_m)
            for t in range(sbt):
                cu = pltpu.async_copy(
                    au_hbm.at[idx_u.at[pl.ds(t * bb, bb)]], buf_u, sem_u)
                cm = pltpu.async_copy(
                    am_hbm.at[idx_m.at[pl.ds(t * bb, bb)]], buf_m, sem_m)
                cu.wait()
                cm.wait()
                pltpu.sync_copy(buf_u, gu_hbm.at[pl.ds(base + t * bb, bb)])
                pltpu.sync_copy(buf_m, gm_hbm.at[pl.ds(base + t * bb, bb)])
            return carry

        lax.fori_loop(0, trips // sbt, superbatch, 0)

    return k(a_u, a_m, ei_u, ei_m)


def kernel(x_user, x_merchant, edge_index_um, edge_index_mu, edge_attr,
           edge_index, W_user, b_user, W_merch, b_merch,
           c1_um_Wl, c1_um_bl, c1_um_Wr, c1_mu_Wl, c1_mu_bl, c1_mu_Wr,
           c2_um_Wl, c2_um_bl, c2_um_Wr, c2_mu_Wl, c2_mu_bl, c2_mu_Wr,
           W_edge, b_edge, W_cls1, b_cls1, W_cls2, b_cls2):
    e_tot = edge_index_um.shape[1]
    epw = e_tot // NW
    cap = -(-epw // B) * B + 16

    zacc = jnp.zeros((STRIPE, H), jnp.float32)
    onesb = jnp.ones((B, H), jnp.float32)

    srcl_um, dstl_um, cnts_um = _bin_edges(edge_index_um[0], edge_index_um[1])
    srcl_mu, dstl_mu, cnts_mu = _bin_edges(edge_index_mu[0], edge_index_mu[1])
    cnt_m = _seg_counts(dstl_um, cnts_um, zacc, onesb, cap)
    cnt_u = _seg_counts(dstl_mu, cnts_mu, zacc, onesb, cap)

    h_u = _linrelu(x_user, W_user, b_user)
    h_m = _linrelu(x_merchant, W_merch, b_merch)

    s_m = _sage_agg(h_u, srcl_um, dstl_um, cnts_um, zacc, cap)
    s_u = _sage_agg(h_m, srcl_mu, dstl_mu, cnts_mu, zacc, cap)
    h_m1 = _sage_linear(s_m, cnt_m, h_m, c1_um_Wl, c1_um_bl, c1_um_Wr)
    h_u1 = _sage_linear(s_u, cnt_u, h_u, c1_mu_Wl, c1_mu_bl, c1_mu_Wr)

    s_m2 = _sage_agg(h_u1, srcl_um, dstl_um, cnts_um, zacc, cap)
    s_u2 = _sage_agg(h_m1, srcl_mu, dstl_mu, cnts_mu, zacc, cap)
    _, a_m = _sage_linear2(s_m2, cnt_m, h_m1, c2_um_Wl, c2_um_bl, c2_um_Wr,
                           W_cls1[H:2 * H, :])
    _, a_u = _sage_linear2(s_u2, cnt_u, h_u1, c2_mu_Wl, c2_mu_bl, c2_mu_Wr,
                           W_cls1[0:H, :])

    g_u, g_m = _head_gather(a_u, a_m, edge_index[0], edge_index[1])
    return _final(g_u, g_m, edge_attr, W_edge, b_edge, W_cls1[2 * H:, :],
                  b_cls1, W_cls2, b_cls2, 2000)


# speedup vs baseline: 1.0190x; 1.0190x over previous
import functools

import jax
import jax.numpy as jnp
from jax import lax
from jax.experimental import pallas as pl
from jax.experimental.pallas import tpu as pltpu
from jax.experimental.pallas import tpu_sc as plsc


H = 128
NC = 2
NS = 16
NW = NC * NS

K = 4
BIN = 12544
NPAD = K * BIN
TRASH = BIN
B = 192
STRIPE = BIN // NS

TC_BLK = NPAD // 16


def _linrelu_body(x_ref, w_ref, b_ref, o_ref):
    o_ref[...] = jax.nn.relu(
        jnp.dot(x_ref[...], w_ref[...], preferred_element_type=jnp.float32)
        + b_ref[...]
    )


def _linrelu(x, w, b):
    d = x.shape[1]
    h = w.shape[1]
    return pl.pallas_call(
        _linrelu_body,
        grid=(NPAD // TC_BLK,),
        in_specs=[
            pl.BlockSpec((TC_BLK, d), lambda i: (i, 0)),
            pl.BlockSpec((d, h), lambda i: (0, 0)),
            pl.BlockSpec((h,), lambda i: (0,)),
        ],
        out_specs=pl.BlockSpec((TC_BLK, h), lambda i: (i, 0)),
        out_shape=jax.ShapeDtypeStruct((NPAD, h), jnp.float32),
    )(x, w, b)


def _sage_linear_body(s_ref, c_ref, xd_ref, wl_ref, bl_ref, wr_ref, o_ref):
    rec = 1.0 / jnp.maximum(c_ref[...][:, :1], 1.0)
    mean = s_ref[...] * rec
    o_ref[...] = jax.nn.relu(
        jnp.dot(mean, wl_ref[...], preferred_element_type=jnp.float32)
        + bl_ref[...]
        + jnp.dot(xd_ref[...], wr_ref[...], preferred_element_type=jnp.float32)
    )


def _sage_linear(s, cnt, x_dst, wl, bl, wr):
    return pl.pallas_call(
        _sage_linear_body,
        grid=(NPAD // TC_BLK,),
        in_specs=[
            pl.BlockSpec((TC_BLK, H), lambda i: (i, 0)),
            pl.BlockSpec((TC_BLK, H), lambda i: (i, 0)),
            pl.BlockSpec((TC_BLK, H), lambda i: (i, 0)),
            pl.BlockSpec((H, H), lambda i: (0, 0)),
            pl.BlockSpec((H,), lambda i: (0,)),
            pl.BlockSpec((H, H), lambda i: (0, 0)),
        ],
        out_specs=pl.BlockSpec((TC_BLK, H), lambda i: (i, 0)),
        out_shape=jax.ShapeDtypeStruct((NPAD, H), jnp.float32),
    )(s, cnt, x_dst, wl, bl, wr)


def _sage_linear2_body(s_ref, c_ref, xd_ref, wl_ref, bl_ref, wr_ref, wa_ref,
                       o_ref, a_ref):
    rec = 1.0 / jnp.maximum(c_ref[...][:, :1], 1.0)
    mean = s_ref[...] * rec
    h2 = jax.nn.relu(
        jnp.dot(mean, wl_ref[...], preferred_element_type=jnp.float32)
        + bl_ref[...]
        + jnp.dot(xd_ref[...], wr_ref[...], preferred_element_type=jnp.float32)
    )
    o_ref[...] = h2
    a_ref[...] = jnp.dot(h2, wa_ref[...], preferred_element_type=jnp.float32)


def _sage_linear2(s, cnt, x_dst, wl, bl, wr, wa):
    return pl.pallas_call(
        _sage_linear2_body,
        grid=(NPAD // TC_BLK,),
        in_specs=[
            pl.BlockSpec((TC_BLK, H), lambda i: (i, 0)),
            pl.BlockSpec((TC_BLK, H), lambda i: (i, 0)),
            pl.BlockSpec((TC_BLK, H), lambda i: (i, 0)),
            pl.BlockSpec((H, H), lambda i: (0, 0)),
            pl.BlockSpec((H,), lambda i: (0,)),
            pl.BlockSpec((H, H), lambda i: (0, 0)),
            pl.BlockSpec((H, H), lambda i: (0, 0)),
        ],
        out_specs=[
            pl.BlockSpec((TC_BLK, H), lambda i: (i, 0)),
            pl.BlockSpec((TC_BLK, H), lambda i: (i, 0)),
        ],
        out_shape=[
            jax.ShapeDtypeStruct((NPAD, H), jnp.float32),
            jax.ShapeDtypeStruct((NPAD, H), jnp.float32),
        ],
    )(s, cnt, x_dst, wl, bl, wr, wa)


def _final_body(gu_ref, gm_ref, ea_ref, we_ref, be_ref, w1c_ref, b1_ref,
                w2_ref, b2_ref, o_ref):
    e = jax.nn.relu(
        jnp.dot(ea_ref[...], we_ref[...], preferred_element_type=jnp.float32)
        + be_ref[...]
    )
    acc = gu_ref[...] + gm_ref[...] + jnp.dot(
        e, w1c_ref[...], preferred_element_type=jnp.float32)
    h = jax.nn.relu(acc + b1_ref[...])
    o_ref[...] = (
        jnp.dot(h, w2_ref[...], preferred_element_type=jnp.float32) + b2_ref[...]
    )


def _final(gu, gm, ea, we, be, w1c, b1, w2, b2, block):
    n = gu.shape[0]
    d_e = ea.shape[1]
    return pl.pallas_call(
        _final_body,
        grid=(n // block,),
        in_specs=[
            pl.BlockSpec((block, H), lambda i: (i, 0)),
            pl.BlockSpec((block, H), lambda i: (i, 0)),
            pl.BlockSpec((block, d_e), lambda i: (i, 0)),
            pl.BlockSpec((d_e, H), lambda i: (0, 0)),
            pl.BlockSpec((H,), lambda i: (0,)),
            pl.BlockSpec((H, H), lambda i: (0, 0)),
            pl.BlockSpec((H,), lambda i: (0,)),
            pl.BlockSpec((H, 2), lambda i: (0, 0)),
            pl.BlockSpec((2,), lambda i: (0,)),
        ],
        out_specs=pl.BlockSpec((block, 2), lambda i: (i, 0)),
        out_shape=jax.ShapeDtypeStruct((n, 2), jnp.float32),
    )(gu, gm, ea, we, be, w1c, b1, w2, b2)


def _sc_mesh():
    return plsc.VectorSubcoreMesh(core_axis_name="c", subcore_axis_name="s",
                                  num_cores=NC, num_subcores=NS)


def _bin_edges(ei_src, ei_dst):
    e_tot = ei_src.shape[0]
    epw = e_tot // NW
    cap = -(-epw // B) * B + 16

    @functools.partial(
        pl.kernel,
        out_type=[
            jax.ShapeDtypeStruct((K * NW * cap,), jnp.int32),
            jax.ShapeDtypeStruct((K * NW * cap,), jnp.int32),
            jax.ShapeDtypeStruct((NW * 16,), jnp.int32),
        ],
        mesh=_sc_mesh(),
        compiler_params=pltpu.CompilerParams(needs_layout_passes=False),
        scratch_types=[
            pltpu.VMEM((epw,), jnp.int32),
            pltpu.VMEM((epw,), jnp.int32),
            [pltpu.VMEM((cap,), jnp.int32) for _ in range(K)],
            [pltpu.VMEM((cap,), jnp.int32) for _ in range(K)],
            pltpu.VMEM((16,), jnp.int32),
        ],
    )
    def k(es_hbm, ed_hbm, srcl_hbm, dstl_hbm, cnts_hbm,
          srcbuf, dstbuf, sbufs, dbufs, countbuf):
        wid = lax.axis_index("s") * NC + lax.axis_index("c")
        lane = lax.iota(jnp.int32, 16)
        pltpu.sync_copy(es_hbm.at[pl.ds(wid * epw, epw)], srcbuf)
        pltpu.sync_copy(ed_hbm.at[pl.ds(wid * epw, epw)], dstbuf)

        def step(i, offs):
            vs = srcbuf[pl.ds(i * 16, 16)]
            vd = dstbuf[pl.ds(i * 16, 16)]
            binv = ((vd >= BIN).astype(jnp.int32)
                    + (vd >= 2 * BIN).astype(jnp.int32)
                    + (vd >= 3 * BIN).astype(jnp.int32))
            dstl = vd - binv * BIN
            new = []
            for kk in range(K):
                m = binv == kk
                o = offs[kk]
                plsc.store_compressed(sbufs[kk].at[pl.ds(o, 16)], vs, mask=m)
                plsc.store_compressed(dbufs[kk].at[pl.ds(o, 16)], dstl, mask=m)
                pc = plsc.all_reduce_population_count(m)
                new.append(o + pc[0])
            return tuple(new)

        offs = lax.fori_loop(0, epw // 16, step, (0, 0, 0, 0))

        cvec = jnp.zeros((16,), jnp.int32)
        for kk in range(K):
            n = offs[kk]
            base = (n >> 4) << 4
            npad = ((n + (B - 1)) // B) * B
            keep = lane < (n - base)
            vs_old = sbufs[kk][pl.ds(base, 16)]
            vd_old = dbufs[kk][pl.ds(base, 16)]
            sbufs[kk][pl.ds(base, 16)] = jnp.where(keep, vs_old, 0)
            dbufs[kk][pl.ds(base, 16)] = jnp.where(
                keep, vd_old, jnp.full((16,), TRASH, jnp.int32))

            def pad_step(j, carry, kk=kk):
                sbufs[kk][pl.ds(j * 16, 16)] = jnp.zeros((16,), jnp.int32)
                dbufs[kk][pl.ds(j * 16, 16)] = jnp.full((16,), TRASH,
                                                        jnp.int32)
                return carry

            lax.fori_loop((base >> 4) + 1, npad >> 4, pad_step, 0)
            cvec = jnp.where(lane == kk, npad // B, cvec)
            off_hbm = (kk * NW + wid) * cap
            pltpu.sync_copy(sbufs[kk].at[pl.ds(0, cap)],
                            srcl_hbm.at[pl.ds(off_hbm, cap)])
            pltpu.sync_copy(dbufs[kk].at[pl.ds(0, cap)],
                            dstl_hbm.at[pl.ds(off_hbm, cap)])
        countbuf[...] = cvec
        pltpu.sync_copy(countbuf, cnts_hbm.at[pl.ds(wid * 16, 16)])

    return k(ei_src, ei_dst)


def _sage_agg(h_src, src_list, dstl_list, counts, zeros_acc, cap):
    @functools.partial(
        pl.kernel,
        out_type=jax.ShapeDtypeStruct((NPAD, H), jnp.float32),
        mesh=_sc_mesh(),
        compiler_params=pltpu.CompilerParams(needs_layout_passes=False),
        scratch_types=[
            pltpu.VMEM((4 * B,), jnp.int32),
            pltpu.VMEM((4 * B,), jnp.int32),
            [pltpu.VMEM((B // 4, H), jnp.float32) for _ in range(4)],
            pltpu.VMEM((NW * 16,), jnp.int32),
            pltpu.VMEM_SHARED((BIN + 16, H), jnp.float32),
            [pltpu.SemaphoreType.DMA for _ in range(4)],
        ],
    )
    def k(h_hbm, srcl_hbm, dstl_hbm, cnts_hbm, zacc_hbm, s_hbm,
          idx0, dl0, gbs, cbuf, acc, sems):
        c = lax.axis_index("c")
        sid = lax.axis_index("s")
        lane = lax.iota(jnp.int32, 16)
        pltpu.sync_copy(cnts_hbm, cbuf)
        sbase = sid * STRIPE

        for ki in range(2):
            kbin = c * 2 + ki
            pltpu.sync_copy(zacc_hbm, acc.at[pl.ds(sbase, STRIPE)])
            plsc.subcore_barrier()
            for li in range(2):
                st = sid * 2 + li
                cvec = cbuf[pl.ds(st * 16, 16)]
                trips = jnp.max(jnp.where(lane == kbin, cvec, 0))
                listbase = (kbin * NW + st) * cap

                G = B // 4

                def run16(ioff, n16):
                    def g_issue(t):
                        pltpu.async_copy(
                            h_hbm.at[idx0.at[pl.ds(t * G, G)]],
                            gbs[t % 4], sems[t % 4])

                    def g_wait(t):
                        pltpu.make_async_copy(
                            h_hbm.at[idx0.at[pl.ds(t * G, G)]],
                            gbs[t % 4], sems[t % 4]).wait()

                    def s_issue(t):
                        pltpu.async_copy(gbs[t % 4],
                                         acc.at[dl0.at[pl.ds(t * G, G)]],
                                         sems[t % 4], add=True)

                    def s_wait(t):
                        pltpu.make_async_copy(
                            gbs[t % 4], acc.at[dl0.at[pl.ds(t * G, G)]],
                            sems[t % 4]).wait()

                    g_issue(0)
                    if n16 > 1:
                        g_issue(1)
                    for t in range(n16):
                        if t + 2 < n16:
                            if t - 2 >= 0:
                                s_wait(t - 2)
                            g_issue(t + 2)
                        g_wait(t)
                        s_issue(t)
                    for t in range(max(n16 - 4, 0), n16):
                        s_wait(t)
                    return

                def superbatch(sb, carry):
                    off = listbase + sb * (4 * B)
                    pltpu.sync_copy(srcl_hbm.at[pl.ds(off, 4 * B)], idx0)
                    pltpu.sync_copy(dstl_hbm.at[pl.ds(off, 4 * B)], dl0)
                    run16(off, 16)
                    return carry

                lax.fori_loop(0, trips // 4, superbatch, 0)

                def tail(j, carry):
                    off = listbase + j * B
                    pltpu.sync_copy(srcl_hbm.at[pl.ds(off, B)],
                                    idx0.at[pl.ds(0, B)])
                    pltpu.sync_copy(dstl_hbm.at[pl.ds(off, B)],
                                    dl0.at[pl.ds(0, B)])
                    run16(off, 4)
                    return carry

                lax.fori_loop((trips // 4) * 4, trips, tail, 0)
            plsc.subcore_barrier()
            pltpu.sync_copy(acc.at[pl.ds(sbase, STRIPE)],
                            s_hbm.at[pl.ds(kbin * BIN + sbase, STRIPE)])
            plsc.subcore_barrier()

    return k(h_src, src_list, dstl_list, counts, zeros_acc)


def _seg_counts(dstl_list, counts, zeros_cnt, ones_b, cap):
    @functools.partial(
        pl.kernel,
        out_type=jax.ShapeDtypeStruct((NPAD, H), jnp.float32),
        mesh=_sc_mesh(),
        compiler_params=pltpu.CompilerParams(needs_layout_passes=False),
        scratch_types=[
            pltpu.VMEM((8 * B,), jnp.int32),
            pltpu.VMEM((B, H), jnp.float32),
            pltpu.VMEM((NW * 16,), jnp.int32),
            pltpu.VMEM_SHARED((BIN + 16, H), jnp.float32),
        ],
    )
    def k(dstl_hbm, cnts_hbm, zcnt_hbm, ones_hbm,
          cnt_hbm, dstlbuf, ones_v, cbuf, cacc):
        c = lax.axis_index("c")
        sid = lax.axis_index("s")
        lane = lax.iota(jnp.int32, 16)
        pltpu.sync_copy(cnts_hbm, cbuf)
        pltpu.sync_copy(ones_hbm, ones_v)
        sbase = sid * STRIPE

        for ki in range(2):
            kbin = c * 2 + ki
            pltpu.sync_copy(zcnt_hbm, cacc.at[pl.ds(sbase, STRIPE)])
            plsc.subcore_barrier()
            for li in range(2):
                st = sid * 2 + li
                cvec = cbuf[pl.ds(st * 16, 16)]
                trips = jnp.max(jnp.where(lane == kbin, cvec, 0))
                listbase = (kbin * NW + st) * cap

                def superbatch(sb, carry):
                    off = listbase + sb * (8 * B)
                    pltpu.sync_copy(dstl_hbm.at[pl.ds(off, 8 * B)], dstlbuf)
                    for t in range(8):
                        pltpu.sync_copy(ones_v,
                                        cacc.at[dstlbuf.at[pl.ds(t * B, B)]],
                                        add=True)
                    return carry

                lax.fori_loop(0, trips // 8, superbatch, 0)

                def tail(j, carry):
                    off = listbase + j * B
                    pltpu.sync_copy(dstl_hbm.at[pl.ds(off, B)],
                                    dstlbuf.at[pl.ds(0, B)])
                    pltpu.sync_copy(ones_v,
                                    cacc.at[dstlbuf.at[pl.ds(0, B)]],
                                    add=True)
                    return carry

                lax.fori_loop((trips // 8) * 8, trips, tail, 0)
            plsc.subcore_barrier()
            pltpu.sync_copy(cacc.at[pl.ds(sbase, STRIPE)],
                            cnt_hbm.at[pl.ds(kbin * BIN + sbase, STRIPE)])
            plsc.subcore_barrier()

    return k(dstl_list, counts, zeros_cnt, ones_b)


def _head_gather(a_u, a_m, ei_u, ei_m):
    e_tot = ei_u.shape[0]
    per_w = e_tot // NW
    bb = 400
    sbt = 5
    trips = per_w // bb
    assert trips % sbt == 0 and per_w % bb == 0

    @functools.partial(
        pl.kernel,
        out_type=[
            jax.ShapeDtypeStruct((e_tot, H), jnp.float32),
            jax.ShapeDtypeStruct((e_tot, H), jnp.float32),
        ],
        mesh=_sc_mesh(),
        compiler_params=pltpu.CompilerParams(needs_layout_passes=False),
        scratch_types=[
            pltpu.VMEM((sbt * bb,), jnp.int32),
            pltpu.VMEM((sbt * bb,), jnp.int32),
            pltpu.VMEM((bb, H), jnp.float32),
            pltpu.VMEM((bb, H), jnp.float32),
            pltpu.SemaphoreType.DMA,
            pltpu.SemaphoreType.DMA,
        ],
    )
    def k(au_hbm, am_hbm, eiu_hbm, eim_hbm, gu_hbm, gm_hbm,
          idx_u, idx_m, buf_u, buf_m, sem_u, sem_m):
        wid = lax.axis_index("s") * NC + lax.axis_index("c")
        base_w = wid * per_w

        def superbatch(sb, carry):
            base = base_w + sb * (sbt * bb)
            pltpu.sync_copy(eiu_hbm.at[pl.ds(base, sbt * bb)], idx_u)
            pltpu.sync_copy(eim_hbm.at[pl.ds(base, sbt * bb)], idx_m)
            for t in range(sbt):
                cu = pltpu.async_copy(
                    au_hbm.at[idx_u.at[pl.ds(t * bb, bb)]], buf_u, sem_u)
                cm = pltpu.async_copy(
                    am_hbm.at[idx_m.at[pl.ds(t * bb, bb)]], buf_m, sem_m)
                cu.wait()
                cm.wait()
                pltpu.sync_copy(buf_u, gu_hbm.at[pl.ds(base + t * bb, bb)])
                pltpu.sync_copy(buf_m, gm_hbm.at[pl.ds(base + t * bb, bb)])
            return carry

        lax.fori_loop(0, trips // sbt, superbatch, 0)

    return k(a_u, a_m, ei_u, ei_m)


def kernel(x_user, x_merchant, edge_index_um, edge_index_mu, edge_attr,
           edge_index, W_user, b_user, W_merch, b_merch,
           c1_um_Wl, c1_um_bl, c1_um_Wr, c1_mu_Wl, c1_mu_bl, c1_mu_Wr,
           c2_um_Wl, c2_um_bl, c2_um_Wr, c2_mu_Wl, c2_mu_bl, c2_mu_Wr,
           W_edge, b_edge, W_cls1, b_cls1, W_cls2, b_cls2):
    e_tot = edge_index_um.shape[1]
    epw = e_tot // NW
    cap = -(-epw // B) * B + 16

    zacc = jnp.zeros((STRIPE, H), jnp.float32)
    onesb = jnp.ones((B, H), jnp.float32)

    srcl_um, dstl_um, cnts_um = _bin_edges(edge_index_um[0], edge_index_um[1])
    srcl_mu, dstl_mu, cnts_mu = _bin_edges(edge_index_mu[0], edge_index_mu[1])
    cnt_m = _seg_counts(dstl_um, cnts_um, zacc, onesb, cap)
    cnt_u = _seg_counts(dstl_mu, cnts_mu, zacc, onesb, cap)

    h_u = _linrelu(x_user, W_user, b_user)
    h_m = _linrelu(x_merchant, W_merch, b_merch)

    s_m = _sage_agg(h_u, srcl_um, dstl_um, cnts_um, zacc, cap)
    s_u = _sage_agg(h_m, srcl_mu, dstl_mu, cnts_mu, zacc, cap)
    h_m1 = _sage_linear(s_m, cnt_m, h_m, c1_um_Wl, c1_um_bl, c1_um_Wr)
    h_u1 = _sage_linear(s_u, cnt_u, h_u, c1_mu_Wl, c1_mu_bl, c1_mu_Wr)

    s_m2 = _sage_agg(h_u1, srcl_um, dstl_um, cnts_um, zacc, cap)
    s_u2 = _sage_agg(h_m1, srcl_mu, dstl_mu, cnts_mu, zacc, cap)
    _, a_m = _sage_linear2(s_m2, cnt_m, h_m1, c2_um_Wl, c2_um_bl, c2_um_Wr,
                           W_cls1[H:2 * H, :])
    _, a_u = _sage_linear2(s_u2, cnt_u, h_u1, c2_mu_Wl, c2_mu_bl, c2_mu_Wr,
                           W_cls1[0:H, :])

    g_u, g_m = _head_gather(a_u, a_m, edge_index[0], edge_index[1])
    return _final(g_u, g_m, edge_attr, W_edge, b_edge, W_cls1[2 * H:, :],
                  b_cls1, W_cls2, b_cls2, 2000)


# speedup vs baseline: 1.3855x; 1.3597x over previous
import functools

import jax
import jax.numpy as jnp
from jax import lax
from jax.experimental import pallas as pl
from jax.experimental.pallas import tpu as pltpu
from jax.experimental.pallas import tpu_sc as plsc


H = 128
NC = 2
NS = 16
NW = NC * NS

K = 4
BIN = 12544
NPAD = K * BIN
TRASH = BIN
B = 96
STRIPE = BIN // NS

TC_BLK = NPAD // 16


def _linrelu_body(x_ref, w_ref, b_ref, o_ref):
    o_ref[...] = jax.nn.relu(
        jnp.dot(x_ref[...], w_ref[...], preferred_element_type=jnp.float32)
        + b_ref[...]
    )


def _linrelu(x, w, b):
    d = x.shape[1]
    h = w.shape[1]
    return pl.pallas_call(
        _linrelu_body,
        grid=(NPAD // TC_BLK,),
        in_specs=[
            pl.BlockSpec((TC_BLK, d), lambda i: (i, 0)),
            pl.BlockSpec((d, h), lambda i: (0, 0)),
            pl.BlockSpec((h,), lambda i: (0,)),
        ],
        out_specs=pl.BlockSpec((TC_BLK, h), lambda i: (i, 0)),
        out_shape=jax.ShapeDtypeStruct((NPAD, h), jnp.float32),
    )(x, w, b)


def _sage_linear_body(s_ref, c_ref, xd_ref, wl_ref, bl_ref, wr_ref, o_ref):
    rec = 1.0 / jnp.maximum(c_ref[...][:, :1], 1.0)
    mean = s_ref[...] * rec
    o_ref[...] = jax.nn.relu(
        jnp.dot(mean, wl_ref[...], preferred_element_type=jnp.float32)
        + bl_ref[...]
        + jnp.dot(xd_ref[...], wr_ref[...], preferred_element_type=jnp.float32)
    )


def _sage_linear(s, cnt, x_dst, wl, bl, wr):
    return pl.pallas_call(
        _sage_linear_body,
        grid=(NPAD // TC_BLK,),
        in_specs=[
            pl.BlockSpec((TC_BLK, H), lambda i: (i, 0)),
            pl.BlockSpec((TC_BLK, H), lambda i: (i, 0)),
            pl.BlockSpec((TC_BLK, H), lambda i: (i, 0)),
            pl.BlockSpec((H, H), lambda i: (0, 0)),
            pl.BlockSpec((H,), lambda i: (0,)),
            pl.BlockSpec((H, H), lambda i: (0, 0)),
        ],
        out_specs=pl.BlockSpec((TC_BLK, H), lambda i: (i, 0)),
        out_shape=jax.ShapeDtypeStruct((NPAD, H), jnp.float32),
    )(s, cnt, x_dst, wl, bl, wr)


def _sage_linear2_body(s_ref, c_ref, xd_ref, wl_ref, bl_ref, wr_ref, wa_ref,
                       o_ref, a_ref):
    rec = 1.0 / jnp.maximum(c_ref[...][:, :1], 1.0)
    mean = s_ref[...] * rec
    h2 = jax.nn.relu(
        jnp.dot(mean, wl_ref[...], preferred_element_type=jnp.float32)
        + bl_ref[...]
        + jnp.dot(xd_ref[...], wr_ref[...], preferred_element_type=jnp.float32)
    )
    o_ref[...] = h2
    a_ref[...] = jnp.dot(h2, wa_ref[...], preferred_element_type=jnp.float32)


def _sage_linear2(s, cnt, x_dst, wl, bl, wr, wa):
    return pl.pallas_call(
        _sage_linear2_body,
        grid=(NPAD // TC_BLK,),
        in_specs=[
            pl.BlockSpec((TC_BLK, H), lambda i: (i, 0)),
            pl.BlockSpec((TC_BLK, H), lambda i: (i, 0)),
            pl.BlockSpec((TC_BLK, H), lambda i: (i, 0)),
            pl.BlockSpec((H, H), lambda i: (0, 0)),
            pl.BlockSpec((H,), lambda i: (0,)),
            pl.BlockSpec((H, H), lambda i: (0, 0)),
            pl.BlockSpec((H, H), lambda i: (0, 0)),
        ],
        out_specs=[
            pl.BlockSpec((TC_BLK, H), lambda i: (i, 0)),
            pl.BlockSpec((TC_BLK, H), lambda i: (i, 0)),
        ],
        out_shape=[
            jax.ShapeDtypeStruct((NPAD, H), jnp.float32),
            jax.ShapeDtypeStruct((NPAD, H), jnp.float32),
        ],
    )(s, cnt, x_dst, wl, bl, wr, wa)


def _final_body(gu_ref, gm_ref, ea_ref, we_ref, be_ref, w1c_ref, b1_ref,
                w2_ref, b2_ref, o_ref):
    e = jax.nn.relu(
        jnp.dot(ea_ref[...], we_ref[...], preferred_element_type=jnp.float32)
        + be_ref[...]
    )
    acc = gu_ref[...] + gm_ref[...] + jnp.dot(
        e, w1c_ref[...], preferred_element_type=jnp.float32)
    h = jax.nn.relu(acc + b1_ref[...])
    o_ref[...] = (
        jnp.dot(h, w2_ref[...], preferred_element_type=jnp.float32) + b2_ref[...]
    )


def _final(gu, gm, ea, we, be, w1c, b1, w2, b2, block):
    n = gu.shape[0]
    d_e = ea.shape[1]
    return pl.pallas_call(
        _final_body,
        grid=(n // block,),
        in_specs=[
            pl.BlockSpec((block, H), lambda i: (i, 0)),
            pl.BlockSpec((block, H), lambda i: (i, 0)),
            pl.BlockSpec((block, d_e), lambda i: (i, 0)),
            pl.BlockSpec((d_e, H), lambda i: (0, 0)),
            pl.BlockSpec((H,), lambda i: (0,)),
            pl.BlockSpec((H, H), lambda i: (0, 0)),
            pl.BlockSpec((H,), lambda i: (0,)),
            pl.BlockSpec((H, 2), lambda i: (0, 0)),
            pl.BlockSpec((2,), lambda i: (0,)),
        ],
        out_specs=pl.BlockSpec((block, 2), lambda i: (i, 0)),
        out_shape=jax.ShapeDtypeStruct((n, 2), jnp.float32),
    )(gu, gm, ea, we, be, w1c, b1, w2, b2)


def _sc_mesh():
    return plsc.VectorSubcoreMesh(core_axis_name="c", subcore_axis_name="s",
                                  num_cores=NC, num_subcores=NS)


def _bin_edges(ei_src, ei_dst):
    e_tot = ei_src.shape[0]
    epw = e_tot // NW
    cap = -(-epw // B) * B + 16

    @functools.partial(
        pl.kernel,
        out_type=[
            jax.ShapeDtypeStruct((K * NW * cap,), jnp.int32),
            jax.ShapeDtypeStruct((K * NW * cap,), jnp.int32),
            jax.ShapeDtypeStruct((NW * 16,), jnp.int32),
        ],
        mesh=_sc_mesh(),
        compiler_params=pltpu.CompilerParams(needs_layout_passes=False),
        scratch_types=[
            pltpu.VMEM((epw,), jnp.int32),
            pltpu.VMEM((epw,), jnp.int32),
            [pltpu.VMEM((cap,), jnp.int32) for _ in range(K)],
            [pltpu.VMEM((cap,), jnp.int32) for _ in range(K)],
            pltpu.VMEM((16,), jnp.int32),
        ],
    )
    def k(es_hbm, ed_hbm, srcl_hbm, dstl_hbm, cnts_hbm,
          srcbuf, dstbuf, sbufs, dbufs, countbuf):
        wid = lax.axis_index("s") * NC + lax.axis_index("c")
        lane = lax.iota(jnp.int32, 16)
        pltpu.sync_copy(es_hbm.at[pl.ds(wid * epw, epw)], srcbuf)
        pltpu.sync_copy(ed_hbm.at[pl.ds(wid * epw, epw)], dstbuf)

        def step(i, offs):
            vs = srcbuf[pl.ds(i * 16, 16)]
            vd = dstbuf[pl.ds(i * 16, 16)]
            binv = ((vd >= BIN).astype(jnp.int32)
                    + (vd >= 2 * BIN).astype(jnp.int32)
                    + (vd >= 3 * BIN).astype(jnp.int32))
            dstl = vd - binv * BIN
            new = []
            for kk in range(K):
                m = binv == kk
                o = offs[kk]
                plsc.store_compressed(sbufs[kk].at[pl.ds(o, 16)], vs, mask=m)
                plsc.store_compressed(dbufs[kk].at[pl.ds(o, 16)], dstl, mask=m)
                pc = plsc.all_reduce_population_count(m)
                new.append(o + pc[0])
            return tuple(new)

        offs = lax.fori_loop(0, epw // 16, step, (0, 0, 0, 0))

        cvec = jnp.zeros((16,), jnp.int32)
        for kk in range(K):
            n = offs[kk]
            base = (n >> 4) << 4
            npad = ((n + (B - 1)) // B) * B
            keep = lane < (n - base)
            vs_old = sbufs[kk][pl.ds(base, 16)]
            vd_old = dbufs[kk][pl.ds(base, 16)]
            sbufs[kk][pl.ds(base, 16)] = jnp.where(keep, vs_old, 0)
            dbufs[kk][pl.ds(base, 16)] = jnp.where(
                keep, vd_old, jnp.full((16,), TRASH, jnp.int32))

            def pad_step(j, carry, kk=kk):
                sbufs[kk][pl.ds(j * 16, 16)] = jnp.zeros((16,), jnp.int32)
                dbufs[kk][pl.ds(j * 16, 16)] = jnp.full((16,), TRASH,
                                                        jnp.int32)
                return carry

            lax.fori_loop((base >> 4) + 1, npad >> 4, pad_step, 0)
            cvec = jnp.where(lane == kk, npad // B, cvec)
            off_hbm = (kk * NW + wid) * cap
            pltpu.sync_copy(sbufs[kk].at[pl.ds(0, cap)],
                            srcl_hbm.at[pl.ds(off_hbm, cap)])
            pltpu.sync_copy(dbufs[kk].at[pl.ds(0, cap)],
                            dstl_hbm.at[pl.ds(off_hbm, cap)])
        countbuf[...] = cvec
        pltpu.sync_copy(countbuf, cnts_hbm.at[pl.ds(wid * 16, 16)])

    return k(ei_src, ei_dst)


def _sage_agg(h_src, src_list, dstl_list, counts, zeros_acc, cap):
    @functools.partial(
        pl.kernel,
        out_type=jax.ShapeDtypeStruct((NPAD, H), jnp.float32),
        mesh=_sc_mesh(),
        compiler_params=pltpu.CompilerParams(needs_layout_passes=False),
        scratch_types=[
            pltpu.VMEM((8 * B,), jnp.int32),
            pltpu.VMEM((B,), jnp.int32),
            pltpu.VMEM((8 * B,), jnp.int32),
            pltpu.VMEM((B,), jnp.int32),
            pltpu.VMEM((B, H), jnp.float32),
            pltpu.VMEM((B, H), jnp.float32),
            pltpu.VMEM((NW * 16,), jnp.int32),
            pltpu.VMEM_SHARED((BIN + 16, H), jnp.float32),
            pltpu.SemaphoreType.DMA,
            pltpu.SemaphoreType.DMA,
            pltpu.SemaphoreType.DMA,
            pltpu.SemaphoreType.DMA,
        ],
    )
    def k(h_hbm, srcl_hbm, dstl_hbm, cnts_hbm, zacc_hbm, s_hbm,
          idx0, idx1, dl0, dl1, gb0, gb1, cbuf, acc, sem0, sem1,
          sems0, sems1):
        c = lax.axis_index("c")
        sid = lax.axis_index("s")
        lane = lax.iota(jnp.int32, 16)
        pltpu.sync_copy(cnts_hbm, cbuf)
        sbase = sid * STRIPE

        for ki in range(2):
            kbin = c * 2 + ki
            pltpu.sync_copy(zacc_hbm, acc.at[pl.ds(sbase, STRIPE)])
            plsc.subcore_barrier()
            for li in range(2):
                st = sid * 2 + li
                cvec = cbuf[pl.ds(st * 16, 16)]
                trips = jnp.max(jnp.where(lane == kbin, cvec, 0))
                listbase = (kbin * NW + st) * cap

                gbs = (gb0, gb1)

                def superbatch(sb, carry):
                    off = listbase + sb * (8 * B)
                    pltpu.sync_copy(srcl_hbm.at[pl.ds(off, 8 * B)], idx0)
                    pltpu.sync_copy(dstl_hbm.at[pl.ds(off, 8 * B)], dl0)
                    pltpu.async_copy(
                        h_hbm.at[idx0.at[pl.ds(0, B)]], gbs[0], sem0)
                    for t in range(8):
                        if t + 1 < 8:
                            if t >= 1:
                                pltpu.make_async_copy(
                                    gbs[(t - 1) & 1],
                                    acc.at[dl0.at[pl.ds((t - 1) * B, B)]],
                                    (sems0, sems1)[(t - 1) & 1]).wait()
                            pltpu.async_copy(
                                h_hbm.at[idx0.at[pl.ds((t + 1) * B, B)]],
                                gbs[(t + 1) & 1], (sem0, sem1)[(t + 1) & 1])
                        pltpu.make_async_copy(
                            h_hbm.at[idx0.at[pl.ds(t * B, B)]],
                            gbs[t & 1], (sem0, sem1)[t & 1]).wait()
                        pltpu.async_copy(gbs[t & 1],
                                         acc.at[dl0.at[pl.ds(t * B, B)]],
                                         (sems0, sems1)[t & 1], add=True)
                    pltpu.make_async_copy(
                        gbs[0], acc.at[dl0.at[pl.ds(6 * B, B)]],
                        sems0).wait()
                    pltpu.make_async_copy(
                        gbs[1], acc.at[dl0.at[pl.ds(7 * B, B)]],
                        sems1).wait()
                    return carry

                lax.fori_loop(0, trips // 8, superbatch, 0)

                def tail(j, carry):
                    off = listbase + j * B
                    pltpu.sync_copy(srcl_hbm.at[pl.ds(off, B)], idx1)
                    pltpu.sync_copy(dstl_hbm.at[pl.ds(off, B)], dl1)
                    pltpu.async_copy(h_hbm.at[idx1], gb0, sem0).wait()
                    pltpu.sync_copy(gb0, acc.at[dl1], add=True)
                    return carry

                lax.fori_loop((trips // 8) * 8, trips, tail, 0)
            plsc.subcore_barrier()
            pltpu.sync_copy(acc.at[pl.ds(sbase, STRIPE)],
                            s_hbm.at[pl.ds(kbin * BIN + sbase, STRIPE)])
            plsc.subcore_barrier()

    return k(h_src, src_list, dstl_list, counts, zeros_acc)


def _seg_counts(dstl_list, counts, zeros_cnt, ones_b, cap):
    @functools.partial(
        pl.kernel,
        out_type=jax.ShapeDtypeStruct((NPAD, H), jnp.float32),
        mesh=_sc_mesh(),
        compiler_params=pltpu.CompilerParams(needs_layout_passes=False),
        scratch_types=[
            pltpu.VMEM((8 * B,), jnp.int32),
            pltpu.VMEM((B, H), jnp.float32),
            pltpu.VMEM((NW * 16,), jnp.int32),
            pltpu.VMEM_SHARED((BIN + 16, H), jnp.float32),
        ],
    )
    def k(dstl_hbm, cnts_hbm, zcnt_hbm, ones_hbm,
          cnt_hbm, dstlbuf, ones_v, cbuf, cacc):
        c = lax.axis_index("c")
        sid = lax.axis_index("s")
        lane = lax.iota(jnp.int32, 16)
        pltpu.sync_copy(cnts_hbm, cbuf)
        pltpu.sync_copy(ones_hbm, ones_v)
        sbase = sid * STRIPE

        for ki in range(2):
            kbin = c * 2 + ki
            pltpu.sync_copy(zcnt_hbm, cacc.at[pl.ds(sbase, STRIPE)])
            plsc.subcore_barrier()
            for li in range(2):
                st = sid * 2 + li
                cvec = cbuf[pl.ds(st * 16, 16)]
                trips = jnp.max(jnp.where(lane == kbin, cvec, 0))
                listbase = (kbin * NW + st) * cap

                def superbatch(sb, carry):
                    off = listbase + sb * (8 * B)
                    pltpu.sync_copy(dstl_hbm.at[pl.ds(off, 8 * B)], dstlbuf)
                    for t in range(8):
                        pltpu.sync_copy(ones_v,
                                        cacc.at[dstlbuf.at[pl.ds(t * B, B)]],
                                        add=True)
                    return carry

                lax.fori_loop(0, trips // 8, superbatch, 0)

                def tail(j, carry):
                    off = listbase + j * B
                    pltpu.sync_copy(dstl_hbm.at[pl.ds(off, B)],
                                    dstlbuf.at[pl.ds(0, B)])
                    pltpu.sync_copy(ones_v,
                                    cacc.at[dstlbuf.at[pl.ds(0, B)]],
                                    add=True)
                    return carry

                lax.fori_loop((trips // 8) * 8, trips, tail, 0)
            plsc.subcore_barrier()
            pltpu.sync_copy(cacc.at[pl.ds(sbase, STRIPE)],
                            cnt_hbm.at[pl.ds(kbin * BIN + sbase, STRIPE)])
            plsc.subcore_barrier()

    return k(dstl_list, counts, zeros_cnt, ones_b)


def _head_gather(a_u, a_m, ei_u, ei_m):
    e_tot = ei_u.shape[0]
    per_w = e_tot // NW
    bb = 400
    sbt = 5
    trips = per_w // bb
    assert trips % sbt == 0 and per_w % bb == 0

    @functools.partial(
        pl.kernel,
        out_type=[
            jax.ShapeDtypeStruct((e_tot, H), jnp.float32),
            jax.ShapeDtypeStruct((e_tot, H), jnp.float32),
        ],
        mesh=_sc_mesh(),
        compiler_params=pltpu.CompilerParams(needs_layout_passes=False),
        scratch_types=[
            pltpu.VMEM((sbt * bb,), jnp.int32),
            pltpu.VMEM((sbt * bb,), jnp.int32),
            pltpu.VMEM((bb, H), jnp.float32),
            pltpu.VMEM((bb, H), jnp.float32),
            pltpu.SemaphoreType.DMA,
            pltpu.SemaphoreType.DMA,
        ],
    )
    def k(au_hbm, am_hbm, eiu_hbm, eim_hbm, gu_hbm, gm_hbm,
          idx_u, idx_m, buf_u, buf_m, sem_u, sem_m):
        wid = lax.axis_index("s") * NC + lax.axis_index("c")
        base_w = wid * per_w

        def superbatch(sb, carry):
            base = base_w + sb * (sbt * bb)
            pltpu.sync_copy(eiu_hbm.at[pl.ds(base, sbt * bb)], idx_u)
            pltpu.sync_copy(eim_hbm.at[pl.ds(base, sbt * bb)], idx_m)
            for t in range(sbt):
                cu = pltpu.async_copy(
                    au_hbm.at[idx_u.at[pl.ds(t * bb, bb)]], buf_u, sem_u)
                cm = pltpu.async_copy(
                    am_hbm.at[idx_m.at[pl.ds(t * bb, bb)]], buf_m, sem_m)
                cu.wait()
                cm.wait()
                pltpu.sync_copy(buf_u, gu_hbm.at[pl.ds(base + t * bb, bb)])
                pltpu.sync_copy(buf_m, gm_hbm.at[pl.ds(base + t * bb, bb)])
            return carry

        lax.fori_loop(0, trips // sbt, superbatch, 0)

    return k(a_u, a_m, ei_u, ei_m)


def kernel(x_user, x_merchant, edge_index_um, edge_index_mu, edge_attr,
           edge_index, W_user, b_user, W_merch, b_merch,
           c1_um_Wl, c1_um_bl, c1_um_Wr, c1_mu_Wl, c1_mu_bl, c1_mu_Wr,
           c2_um_Wl, c2_um_bl, c2_um_Wr, c2_mu_Wl, c2_mu_bl, c2_mu_Wr,
           W_edge, b_edge, W_cls1, b_cls1, W_cls2, b_cls2):
    e_tot = edge_index_um.shape[1]
    epw = e_tot // NW
    cap = -(-epw // B) * B + 16

    zacc = jnp.zeros((STRIPE, H), jnp.float32)
    onesb = jnp.ones((B, H), jnp.float32)

    srcl_um, dstl_um, cnts_um = _bin_edges(edge_index_um[0], edge_index_um[1])
    srcl_mu, dstl_mu, cnts_mu = _bin_edges(edge_index_mu[0], edge_index_mu[1])
    cnt_m = _seg_counts(dstl_um, cnts_um, zacc, onesb, cap)
    cnt_u = _seg_counts(dstl_mu, cnts_mu, zacc, onesb, cap)

    h_u = _linrelu(x_user, W_user, b_user)
    h_m = _linrelu(x_merchant, W_merch, b_merch)

    s_m = _sage_agg(h_u, srcl_um, dstl_um, cnts_um, zacc, cap)
    s_u = _sage_agg(h_m, srcl_mu, dstl_mu, cnts_mu, zacc, cap)
    h_m1 = _sage_linear(s_m, cnt_m, h_m, c1_um_Wl, c1_um_bl, c1_um_Wr)
    h_u1 = _sage_linear(s_u, cnt_u, h_u, c1_mu_Wl, c1_mu_bl, c1_mu_Wr)

    s_m2 = _sage_agg(h_u1, srcl_um, dstl_um, cnts_um, zacc, cap)
    s_u2 = _sage_agg(h_m1, srcl_mu, dstl_mu, cnts_mu, zacc, cap)
    _, a_m = _sage_linear2(s_m2, cnt_m, h_m1, c2_um_Wl, c2_um_bl, c2_um_Wr,
                           W_cls1[H:2 * H, :])
    _, a_u = _sage_linear2(s_u2, cnt_u, h_u1, c2_mu_Wl, c2_mu_bl, c2_mu_Wr,
                           W_cls1[0:H, :])

    g_u, g_m = _head_gather(a_u, a_m, edge_index[0], edge_index[1])
    return _final(g_u, g_m, edge_attr, W_edge, b_edge, W_cls1[2 * H:, :],
                  b_cls1, W_cls2, b_cls2, 2000)
